# baseline (speedup 1.0000x reference)
import functools

import jax
import jax.numpy as jnp
from jax import lax
from jax.experimental import pallas as pl
from jax.experimental.pallas import tpu as pltpu
from jax.experimental.pallas import tpu_sc as plsc

NUM_CORES = 2
NUM_SUBCORES = 16
LANES = 16
CHUNK = 128


def _forest_gather(vectors, labels_i32, row_idx, feat_idx):
    T, S = row_idx.shape
    N, D = vectors.shape
    F = feat_idx.shape[1]
    NW = NUM_CORES * NUM_SUBCORES
    assert F == LANES, "feature subset must match SC lane count"
    assert (T * S) % NW == 0
    per_w = (T * S) // NW
    assert NW % T == 0 and S % per_w == 0
    w_per_tree = NW // T
    assert per_w % CHUNK == 0
    nch = per_w // CHUNK

    mesh = plsc.VectorSubcoreMesh(
        core_axis_name="c", subcore_axis_name="s",
        num_cores=NUM_CORES, num_subcores=NUM_SUBCORES)

    @functools.partial(
        pl.kernel,
        out_type=[
            jax.ShapeDtypeStruct((T, S, F), jnp.float32),
            jax.ShapeDtypeStruct((T, S), jnp.int32),
        ],
        mesh=mesh,
        compiler_params=pltpu.CompilerParams(
            needs_layout_passes=False,
            disable_bounds_checks=True,
            disable_semaphore_checks=True,
        ),
        scratch_types=[
            pltpu.VMEM((per_w,), jnp.int32),
            pltpu.VMEM((F,), jnp.int32),
            pltpu.VMEM((2, CHUNK, 256), jnp.float32),
            pltpu.VMEM((4, CHUNK), jnp.int32),
            pltpu.VMEM((2, CHUNK, 16), jnp.float32),
            pltpu.SemaphoreType.DMA((2,)),
            pltpu.SemaphoreType.DMA((4,)),
            pltpu.SemaphoreType.DMA((2,)),
            pltpu.SemaphoreType.DMA((4,)),
        ],
    )
    def run(vec_hbm, lab_hbm, ri_hbm, fi_hbm, out_hbm, olab_hbm,
            idx_v, feat_v, rbuf, lbuf, obuf, sem_r, sem_l, sem_so, sem_sl):
        wid = lax.axis_index("s") * NUM_CORES + lax.axis_index("c")
        t = wid // w_per_tree
        base = (wid % w_per_tree) * per_w

        pltpu.sync_copy(ri_hbm.at[t, pl.ds(base, per_w)], idx_v)
        pltpu.sync_copy(fi_hbm.at[t], feat_v)
        feat = feat_v[:]

        def start_gathers(c):
            p2, p4 = c % 2, c % 4
            idxs = idx_v.at[pl.ds(c * CHUNK, CHUNK)]
            pltpu.async_copy(vec_hbm.at[idxs], rbuf.at[p2], sem_r.at[p2])
            pltpu.async_copy(lab_hbm.at[idxs], lbuf.at[p4], sem_l.at[p4])

        def wait_feat_store(c):
            p2 = c % 2
            pltpu.make_async_copy(
                obuf.at[p2], out_hbm.at[t, pl.ds(base + c * CHUNK, CHUNK), :],
                sem_so.at[p2]).wait()

        def wait_label_store(c):
            p4 = c % 4
            pltpu.make_async_copy(
                lbuf.at[p4], olab_hbm.at[t, pl.ds(base + c * CHUNK, CHUNK)],
                sem_sl.at[p4]).wait()

        start_gathers(0)

        def chunk_body(c, carry):
            p2, p4 = c % 2, c % 4

            @pl.when(c >= 3)
            def _():
                wait_label_store(c - 3)

            @pl.when(c + 1 < nch)
            def _():
                start_gathers(c + 1)

            idxs = idx_v.at[pl.ds(c * CHUNK, CHUNK)]
            pltpu.make_async_copy(vec_hbm.at[idxs], rbuf.at[p2], sem_r.at[p2]).wait()
            pltpu.make_async_copy(lab_hbm.at[idxs], lbuf.at[p4], sem_l.at[p4]).wait()

            @pl.when(c >= 2)
            def _():
                wait_feat_store(c - 2)

            p16 = jnp.full((LANES,), p2, jnp.int32)

            def sub(r, carry2):
                r16 = jnp.full((LANES,), r, jnp.int32)
                obuf[p2, r, :] = plsc.load_gather(rbuf, [p16, r16, feat])
                return carry2

            lax.fori_loop(0, CHUNK, sub, None)

            pltpu.async_copy(
                obuf.at[p2], out_hbm.at[t, pl.ds(base + c * CHUNK, CHUNK), :],
                sem_so.at[p2])
            pltpu.async_copy(
                lbuf.at[p4], olab_hbm.at[t, pl.ds(base + c * CHUNK, CHUNK)],
                sem_sl.at[p4])
            return carry

        lax.fori_loop(0, nch, chunk_body, None)
        wait_feat_store(nch - 2)
        wait_feat_store(nch - 1)
        wait_label_store(nch - 3)
        wait_label_store(nch - 2)
        wait_label_store(nch - 1)

    return run(vectors, labels_i32, row_idx, feat_idx)


def kernel(vectors, labels, row_indices, feat_indices):
    featured, lab = _forest_gather(
        vectors,
        labels.astype(jnp.int32),
        row_indices.astype(jnp.int32),
        feat_indices.astype(jnp.int32),
    )
    return featured, lab.astype(labels.dtype)


# speedup vs baseline: 1.0110x; 1.0110x over previous
import functools

import jax
import jax.numpy as jnp
from jax import lax
from jax.experimental import pallas as pl
from jax.experimental.pallas import tpu as pltpu
from jax.experimental.pallas import tpu_sc as plsc

NUM_CORES = 2
NUM_SUBCORES = 16
LANES = 16
CHUNK = 128


def _forest_gather(vectors, labels_i32, row_idx, feat_idx):
    T, S = row_idx.shape
    N, D = vectors.shape
    F = feat_idx.shape[1]
    NW = NUM_CORES * NUM_SUBCORES
    assert F == LANES, "feature subset must match SC lane count"
    assert (T * S) % NW == 0
    per_w = (T * S) // NW
    assert NW % T == 0 and S % per_w == 0
    w_per_tree = NW // T
    assert per_w % CHUNK == 0
    nch = per_w // CHUNK

    mesh = plsc.VectorSubcoreMesh(
        core_axis_name="c", subcore_axis_name="s",
        num_cores=NUM_CORES, num_subcores=NUM_SUBCORES)

    @functools.partial(
        pl.kernel,
        out_type=[
            jax.ShapeDtypeStruct((T, S, F), jnp.float32),
            jax.ShapeDtypeStruct((T, S), jnp.int32),
        ],
        mesh=mesh,
        compiler_params=pltpu.CompilerParams(
            needs_layout_passes=False,
            disable_bounds_checks=True,
            disable_semaphore_checks=True,
        ),
        scratch_types=[
            pltpu.VMEM((per_w,), jnp.int32),
            pltpu.VMEM((F,), jnp.int32),
            pltpu.VMEM((2, CHUNK, 256), jnp.float32),
            pltpu.VMEM((4, CHUNK), jnp.int32),
            pltpu.VMEM((2, CHUNK, 16), jnp.float32),
            pltpu.SemaphoreType.DMA((2,)),
            pltpu.SemaphoreType.DMA((4,)),
            pltpu.SemaphoreType.DMA((2,)),
            pltpu.SemaphoreType.DMA((4,)),
        ],
    )
    def run(vec_hbm, lab_hbm, ri_hbm, fi_hbm, out_hbm, olab_hbm,
            idx_v, feat_v, rbuf, lbuf, obuf, sem_r, sem_l, sem_so, sem_sl):
        wid = lax.axis_index("s") * NUM_CORES + lax.axis_index("c")
        t = wid // w_per_tree
        base = (wid % w_per_tree) * per_w

        pltpu.sync_copy(ri_hbm.at[t, pl.ds(base, per_w)], idx_v)
        pltpu.sync_copy(fi_hbm.at[t], feat_v)
        feat = feat_v[:]

        def start_gathers(c):
            p2, p4 = c % 2, c % 4
            idxs = idx_v.at[pl.ds(c * CHUNK, CHUNK)]
            pltpu.async_copy(vec_hbm.at[idxs], rbuf.at[p2], sem_r.at[p2])
            pltpu.async_copy(lab_hbm.at[idxs], lbuf.at[p4], sem_l.at[p4])

        def wait_feat_store(c):
            p2 = c % 2
            pltpu.make_async_copy(
                vec_hbm.at[pl.ds(0, (CHUNK * F) // 256), :],
                rbuf.at[p2, pl.ds(0, (CHUNK * F) // 256), :],
                sem_so.at[p2]).wait()

        def wait_label_store(c):
            p4 = c % 4
            pltpu.make_async_copy(
                lbuf.at[p4], olab_hbm.at[t, pl.ds(base + c * CHUNK, CHUNK)],
                sem_sl.at[p4]).wait()

        start_gathers(0)

        def chunk_body(c, carry):
            p2, p4 = c % 2, c % 4

            @pl.when(c >= 3)
            def _():
                wait_label_store(c - 3)

            @pl.when(c + 1 < nch)
            def _():
                start_gathers(c + 1)

            idxs = idx_v.at[pl.ds(c * CHUNK, CHUNK)]
            pltpu.make_async_copy(vec_hbm.at[idxs], rbuf.at[p2], sem_r.at[p2]).wait()
            pltpu.make_async_copy(lab_hbm.at[idxs], lbuf.at[p4], sem_l.at[p4]).wait()

            @pl.when(c >= 2)
            def _():
                wait_feat_store(c - 2)

            p16 = jnp.full((LANES,), p2, jnp.int32)

            def sub(r, carry2):
                r16 = jnp.full((LANES,), r, jnp.int32)
                obuf[p2, r, :] = plsc.load_gather(rbuf, [p16, r16, feat])
                pltpu.async_copy(
                    obuf.at[p2, r, :], out_hbm.at[t, base + c * CHUNK + r, :],
                    sem_so.at[p2])
                return carry2

            lax.fori_loop(0, CHUNK, sub, None)
            pltpu.async_copy(
                lbuf.at[p4], olab_hbm.at[t, pl.ds(base + c * CHUNK, CHUNK)],
                sem_sl.at[p4])
            return carry

        lax.fori_loop(0, nch, chunk_body, None)
        wait_feat_store(nch - 2)
        wait_feat_store(nch - 1)
        wait_label_store(nch - 3)
        wait_label_store(nch - 2)
        wait_label_store(nch - 1)

    return run(vectors, labels_i32, row_idx, feat_idx)


def kernel(vectors, labels, row_indices, feat_indices):
    featured, lab = _forest_gather(
        vectors,
        labels.astype(jnp.int32),
        row_indices.astype(jnp.int32),
        feat_indices.astype(jnp.int32),
    )
    return featured, lab.astype(labels.dtype)
